# Initial kernel scaffold; baseline (speedup 1.0000x reference)
import jax
import jax.numpy as jnp
from jax.experimental import pallas as pl


def kernel(x, importance, W1, b1, W2, b2):
    raise NotImplementedError("write your pallas kernel here")


# speedup vs baseline: 70.8762x; 70.8762x over previous
import jax
import jax.numpy as jnp
from jax.experimental import pallas as pl

CAPACITY = 100000


def _mem_consolidation_kernel(x_ref, w1t_ref, b1_ref, w2t_ref, b2_ref, out_ref):
    x = x_ref[...]
    B = x.shape[0]

    s = jax.lax.dot_general(
        x, x,
        dimension_numbers=(((1,), (1,)), ((), ())),
        preferred_element_type=jnp.float32,
    )

    m = jnp.maximum(jnp.max(s, axis=1, keepdims=True), 0.0)
    e = jnp.exp(s - m)
    denom = jnp.sum(e, axis=1, keepdims=True) + (CAPACITY - B) * jnp.exp(-m)

    num = jax.lax.dot_general(
        e, x,
        dimension_numbers=(((1,), (0,)), ((), ())),
        preferred_element_type=jnp.float32,
    )
    r = num / denom

    h = jax.lax.dot_general(
        r, w1t_ref[...],
        dimension_numbers=(((1,), (0,)), ((), ())),
        preferred_element_type=jnp.float32,
    ) + b1_ref[...]
    h = jnp.maximum(h, 0.0)
    g = jax.lax.dot_general(
        h, w2t_ref[...],
        dimension_numbers=(((1,), (0,)), ((), ())),
        preferred_element_type=jnp.float32,
    ) + b2_ref[...]
    out_ref[...] = x + jax.nn.sigmoid(g)


@jax.jit
def kernel(x, importance, W1, b1, W2, b2):
    del importance
    B, H = x.shape
    w1t = W1.T
    w2t = W2.T
    b1_2d = b1.reshape(1, -1)
    b2_2d = b2.reshape(1, -1)
    return pl.pallas_call(
        _mem_consolidation_kernel,
        out_shape=jax.ShapeDtypeStruct((B, H), x.dtype),
    )(x, w1t, b1_2d, w2t, b2_2d)
